# baseline (speedup 1.0000x reference)
import functools

import jax
import jax.numpy as jnp
from jax import lax
from jax.experimental import pallas as pl
from jax.experimental.pallas import tpu as pltpu
from jax.experimental.pallas import tpu_sc as plsc

_NC = 2
_NS = 16
_SUB = 80


def _build(n_edges, vocab, emb, chunk, interpret=False):
    nw = _NC * _NS
    b_per_w = n_edges // nw
    n_chunks = b_per_w // chunk
    n_sub = chunk // _SUB
    assert n_chunks % 2 == 0 and n_sub * _SUB == chunk
    assert nw * b_per_w == n_edges

    mesh = plsc.VectorSubcoreMesh(core_axis_name="c", subcore_axis_name="s")

    @functools.partial(
        pl.kernel,
        out_type=jax.ShapeDtypeStruct((n_edges, emb), jnp.float32),
        mesh=mesh,
        scratch_types=[
            pltpu.VMEM((chunk,), jnp.int32),
            pltpu.VMEM((chunk,), jnp.int32),
            pltpu.VMEM((chunk, emb), jnp.float32),
            pltpu.VMEM((chunk, emb), jnp.float32),
            pltpu.SemaphoreType.DMA,
            pltpu.SemaphoreType.DMA,
            pltpu.SemaphoreType.DMA,
            pltpu.SemaphoreType.DMA,
            pltpu.SemaphoreType.DMA,
        ],
        interpret=interpret,
        compiler_params=pltpu.CompilerParams(
            needs_layout_passes=False, use_tc_tiling_on_sc=False
        ),
    )
    def k(idx_hbm, table_hbm, out_hbm, idx_a, idx_b, rows_a, rows_b,
          sia, sib, sg, soa, sob):
        wid = lax.axis_index("s") * _NC + lax.axis_index("c")
        base_w = wid * b_per_w

        def idx_src(c):
            return idx_hbm.at[pl.ds(base_w + c * chunk, chunk)]

        def out_dst(c):
            return out_hbm.at[pl.ds(base_w + c * chunk, chunk), :]

        def fire_gathers(idx_v, rows_v):
            def fire(s, carry):
                pltpu.async_copy(
                    table_hbm.at[idx_v.at[pl.ds(s * _SUB, _SUB)]],
                    rows_v.at[pl.ds(s * _SUB, _SUB), :],
                    sg,
                )
                return carry

            lax.fori_loop(0, n_sub, fire, 0)

        def drain_gathers(idx_v, rows_v):
            def drain(s, carry):
                pltpu.make_async_copy(
                    table_hbm.at[idx_v.at[pl.ds(s * _SUB, _SUB)]],
                    rows_v.at[pl.ds(s * _SUB, _SUB), :],
                    sg,
                ).wait()
                return carry

            lax.fori_loop(0, n_sub, drain, 0)

        pltpu.async_copy(idx_src(0), idx_a, sia)
        pltpu.async_copy(idx_src(1), idx_b, sib)

        def pair_body(t, carry):
            c0 = t * 2
            c1 = c0 + 1

            pltpu.make_async_copy(idx_src(c0), idx_a, sia).wait()

            @pl.when(t > 0)
            def _():
                pltpu.make_async_copy(rows_a, out_dst(c0), soa).wait()

            fire_gathers(idx_a, rows_a)
            drain_gathers(idx_a, rows_a)

            @pl.when(t < n_chunks // 2 - 1)
            def _():
                pltpu.async_copy(idx_src(c0 + 2), idx_a, sia)

            pltpu.async_copy(rows_a, out_dst(c0), soa)

            pltpu.make_async_copy(idx_src(c1), idx_b, sib).wait()

            @pl.when(t > 0)
            def _():
                pltpu.make_async_copy(rows_b, out_dst(c1), sob).wait()

            fire_gathers(idx_b, rows_b)
            drain_gathers(idx_b, rows_b)

            @pl.when(t < n_chunks // 2 - 1)
            def _():
                pltpu.async_copy(idx_src(c1 + 2), idx_b, sib)

            pltpu.async_copy(rows_b, out_dst(c1), sob)
            return carry

        lax.fori_loop(0, n_chunks // 2, pair_body, 0)

        pltpu.make_async_copy(rows_a, out_dst(n_chunks - 2), soa).wait()
        pltpu.make_async_copy(rows_b, out_dst(n_chunks - 1), sob).wait()

    return k


def kernel(edge_attr, edge_embedding_weight):
    n_edges = edge_attr.shape[0]
    vocab, emb = edge_embedding_weight.shape
    k = _build(n_edges, vocab, emb, chunk=2000)
    return k(edge_attr.astype(jnp.int32), edge_embedding_weight)


# speedup vs baseline: 3.3396x; 3.3396x over previous
import functools

import jax
import jax.numpy as jnp
from jax import lax
from jax.experimental import pallas as pl
from jax.experimental.pallas import tpu as pltpu
from jax.experimental.pallas import tpu_sc as plsc

_NC = 2
_NS = 16
_L = 16

_BCAST_DNUMS = lax.GatherDimensionNumbers(
    offset_dims=(), collapsed_slice_dims=(0,), start_index_map=(0,)
)


def _bcast_lane(vec, i):
    idx = jnp.full((_L, 1), i, jnp.int32)
    return lax.gather(
        vec, idx, _BCAST_DNUMS, (1,),
        mode=lax.GatherScatterMode.PROMISE_IN_BOUNDS,
    )


def _build(n_edges, vocab, emb, chunk, interpret=False):
    nw = _NC * _NS
    b_per_w = n_edges // nw
    n_chunks = b_per_w // chunk
    groups = chunk // _L
    assert n_chunks % 2 == 0 and groups * _L == chunk
    assert nw * b_per_w == n_edges and emb == _L

    mesh = plsc.VectorSubcoreMesh(core_axis_name="c", subcore_axis_name="s")

    @functools.partial(
        pl.kernel,
        out_type=jax.ShapeDtypeStruct((n_edges * emb,), jnp.float32),
        mesh=mesh,
        scratch_types=[
            pltpu.VMEM((vocab * emb,), jnp.float32),
            pltpu.VMEM((chunk,), jnp.int32),
            pltpu.VMEM((chunk,), jnp.int32),
            pltpu.VMEM((chunk * emb,), jnp.float32),
            pltpu.VMEM((chunk * emb,), jnp.float32),
            pltpu.SemaphoreType.DMA,
            pltpu.SemaphoreType.DMA,
            pltpu.SemaphoreType.DMA,
            pltpu.SemaphoreType.DMA,
        ],
        interpret=interpret,
        compiler_params=pltpu.CompilerParams(needs_layout_passes=False),
    )
    def k(idx_hbm, table_hbm, out_hbm, table_v, idx_a, idx_b, rows_a, rows_b,
          sia, sib, soa, sob):
        wid = lax.axis_index("s") * _NC + lax.axis_index("c")
        base_w = wid * b_per_w
        lane = lax.iota(jnp.int32, _L)

        pltpu.sync_copy(table_hbm, table_v)

        def idx_src(c):
            return idx_hbm.at[pl.ds(base_w + c * chunk, chunk)]

        def out_dst(c):
            return out_hbm.at[pl.ds((base_w + c * chunk) * emb, chunk * emb)]

        def compute_chunk(idx_v, rows_v):
            @plsc.parallel_loop(0, groups, unroll=2)
            def _(g):
                off = pl.multiple_of(g * _L, _L)
                srcb = idx_v[pl.ds(off, _L)] * emb
                for i in range(_L):
                    addr = _bcast_lane(srcb, i) + lane
                    row = plsc.load_gather(table_v, [addr])
                    rows_v[pl.ds((off + i) * emb, emb)] = row

        pltpu.async_copy(idx_src(0), idx_a, sia)
        pltpu.async_copy(idx_src(1), idx_b, sib)

        def pair_body(t, carry):
            c0 = t * 2
            c1 = c0 + 1

            pltpu.make_async_copy(idx_src(c0), idx_a, sia).wait()

            @pl.when(t > 0)
            def _():
                pltpu.make_async_copy(rows_a, out_dst(c0), soa).wait()

            compute_chunk(idx_a, rows_a)

            @pl.when(t < n_chunks // 2 - 1)
            def _():
                pltpu.async_copy(idx_src(c0 + 2), idx_a, sia)

            pltpu.async_copy(rows_a, out_dst(c0), soa)

            pltpu.make_async_copy(idx_src(c1), idx_b, sib).wait()

            @pl.when(t > 0)
            def _():
                pltpu.make_async_copy(rows_b, out_dst(c1), sob).wait()

            compute_chunk(idx_b, rows_b)

            @pl.when(t < n_chunks // 2 - 1)
            def _():
                pltpu.async_copy(idx_src(c1 + 2), idx_b, sib)

            pltpu.async_copy(rows_b, out_dst(c1), sob)
            return carry

        lax.fori_loop(0, n_chunks // 2, pair_body, 0)

        pltpu.make_async_copy(rows_a, out_dst(n_chunks - 2), soa).wait()
        pltpu.make_async_copy(rows_b, out_dst(n_chunks - 1), sob).wait()

    return k


def kernel(edge_attr, edge_embedding_weight):
    n_edges = edge_attr.shape[0]
    vocab, emb = edge_embedding_weight.shape
    k = _build(n_edges, vocab, emb, chunk=2000)
    flat = k(edge_attr.astype(jnp.int32), edge_embedding_weight.reshape(-1))
    return flat.reshape(n_edges, emb)
